# baseline (speedup 1.0000x reference)
import functools

import jax
import jax.numpy as jnp
from jax import lax
from jax.experimental import pallas as pl
from jax.experimental.pallas import tpu as pltpu
from jax.experimental.pallas import tpu_sc as plsc

N = 10000
E = 160000
H = 3
C = 160
HC = 480
B = 64
NPAD = 10240
NRANGE = 8
RANGE = NPAD // NRANGE
NSUB = 16
RPS = RANGE // NSUB
EPS = E // NSUB
EBLK = 2000
ACCW = 496

_mesh = plsc.VectorSubcoreMesh(core_axis_name="c", subcore_axis_name="s")
_sc_params = pltpu.CompilerParams(use_tc_tiling_on_sc=False,
                                  needs_layout_passes=False)


def _lane():
    return lax.iota(jnp.int32, 16)


def _att_evecs(xlb, xrb, att_v, lane):
    def row_body(j, es):
        jf = jnp.full((16,), j, jnp.int32)
        new = []
        for h in range(H):
            acc = jnp.zeros((16,), jnp.float32)
            for k in range(C // 16):
                col = h * C + k * 16
                xlc = plsc.load_gather(xlb, [jf, col + lane])
                xrc = plsc.load_gather(xrb, [jf, col + lane])
                s = xlc + xrc
                m = jnp.maximum(s, 0.2 * s)
                acc = acc + m * att_v[pl.ds(col, 16)]
            eh = jnp.sum(acc)
            new.append(jnp.where(lane == j, eh, es[h]))
        return tuple(new)

    z = jnp.zeros((16,), jnp.float32)
    return lax.fori_loop(0, 16, row_body, (z, z, z))


def _zero_block(ref, width):
    lane = _lane()
    z = jnp.zeros((16,), jnp.float32)

    def body(j, _):
        jf = jnp.full((16,), j, jnp.int32)
        for k in range(width // 16):
            plsc.store_scatter(ref, [jf, k * 16 + lane], z)
        return 0

    lax.fori_loop(0, 16, body, 0)


def _compact_edges(src_v, dst_v, ew_v, ssrc, sdst, sew, lo, hi):
    def body(i, cnt):
        off = i * 16
        s16 = src_v[pl.ds(off, 16)]
        d16 = dst_v[pl.ds(off, 16)]
        w16 = ew_v[pl.ds(off, 16)]
        m = (d16 >= lo) & (d16 < hi)
        plsc.store_compressed(ssrc.at[pl.ds(cnt, 16)], s16, mask=m)
        plsc.store_compressed(sdst.at[pl.ds(cnt, 16)], d16, mask=m)
        plsc.store_compressed(sew.at[pl.ds(cnt, 16)], w16, mask=m)
        pc = plsc.all_reduce_population_count(m)
        return cnt + jnp.max(pc)

    cnt = lax.fori_loop(0, EBLK // 16, body, jnp.int32(0))
    ssrc[pl.ds(cnt, 16)] = jnp.zeros((16,), jnp.int32)
    sdst[pl.ds(cnt, 16)] = jnp.full((16,), lo, jnp.int32)
    sew[pl.ds(cnt, 16)] = jnp.zeros((16,), jnp.float32)
    return cnt


def _gat_sc(src_hbm, dst_hbm, ew_hbm, xl_hbm, xr_hbm, att_hbm, bgat_hbm,
            h1_hbm, deg_hbm,
            src_v, dst_v, ew_v, ssrc, sdst, sew, att_v, bgat_v,
            idx_s, idx_d, idx_w, ebuf, rbuf, xlb, xrb, outb, facc, fh1,
            fdeg, zrow, acc_sh, sem1, sem2):
    cid = lax.axis_index("c")
    sid = lax.axis_index("s")
    lane = _lane()
    pltpu.sync_copy(att_hbm, att_v)
    pltpu.sync_copy(bgat_hbm, bgat_v)
    _zero_block(zrow, ACCW)
    z = jnp.zeros((16,), jnp.float32)
    for c in range(484, ACCW):
        plsc.store_scatter(outb, [lane, jnp.full((16,), c, jnp.int32)], z)

    def rpass(rp, _):
        lo = (cid * (NRANGE // 2) + rp) * RANGE

        def clr(b, _):
            pltpu.sync_copy(zrow, acc_sh.at[pl.ds(sid * RPS + b * 16, 16)])
            return 0

        lax.fori_loop(0, RPS // 16, clr, 0)
        plsc.subcore_barrier()

        def eblock(blk, _):
            e0 = sid * EPS + blk * EBLK
            pltpu.sync_copy(src_hbm.at[pl.ds(e0, EBLK)], src_v)
            pltpu.sync_copy(dst_hbm.at[pl.ds(e0, EBLK)], dst_v)
            pltpu.sync_copy(ew_hbm.at[pl.ds(e0, EBLK)], ew_v)
            cnt = _compact_edges(src_v, dst_v, ew_v, ssrc, sdst, sew,
                                 lo, lo + RANGE)
            ngroups = (cnt + 15) >> 4

            def group(g, _):
                off = g * 16
                s16 = ssrc[pl.ds(off, 16)]
                d16 = sdst[pl.ds(off, 16)]
                w16 = sew[pl.ds(off, 16)]
                valid = (off + lane) < cnt
                idx_s[...] = s16
                idx_d[...] = d16
                cp1 = pltpu.async_copy(xl_hbm.at[idx_s], xlb, sem1)
                cp2 = pltpu.async_copy(xr_hbm.at[idx_d], xrb, sem2)
                cp1.wait()
                cp2.wait()
                ev = _att_evecs(xlb, xrb, att_v, lane)
                vf = jnp.where(valid, 1.0, 0.0).astype(jnp.float32)
                exs = [jnp.exp(ev[h]) * vf for h in range(H)]
                for h in range(H):
                    ebuf[pl.ds(h * 16, 16)] = exs[h]
                    plsc.store_scatter(
                        outb, [lane, jnp.full((16,), 480 + h, jnp.int32)],
                        exs[h])
                plsc.store_scatter(
                    outb, [lane, jnp.full((16,), 483, jnp.int32)], w16 * vf)

                def edge(j, _):
                    jf = jnp.full((16,), j, jnp.int32)
                    for h in range(H):
                        exj = plsc.load_gather(
                            ebuf, [jnp.full((16,), h * 16 + j, jnp.int32)])
                        for k in range(C // 16):
                            col = h * C + k * 16
                            xlc = plsc.load_gather(xlb, [jf, col + lane])
                            plsc.store_scatter(outb, [jf, col + lane],
                                               xlc * exj)
                    return 0

                lax.fori_loop(0, 16, edge, 0)
                idx_w[...] = jnp.where(valid, d16 - lo, 0)
                pltpu.sync_copy(outb, acc_sh.at[idx_w], add=True)
                return 0

            lax.fori_loop(0, ngroups, group, 0)
            return 0

        lax.fori_loop(0, EPS // EBLK, eblock, 0)
        plsc.subcore_barrier()

        def fblock(b, _):
            r0l = sid * RPS + b * 16
            r0g = lo + r0l
            pltpu.sync_copy(xl_hbm.at[pl.ds(r0g, 16)], xlb)
            pltpu.sync_copy(xr_hbm.at[pl.ds(r0g, 16)], xrb)
            pltpu.sync_copy(acc_sh.at[pl.ds(r0l, 16)], facc)
            ev = _att_evecs(xlb, xrb, att_v, lane)
            for h in range(H):
                exh = jnp.exp(ev[h])
                den = plsc.load_gather(
                    facc, [lane, jnp.full((16,), 480 + h, jnp.int32)]) + exh
                ebuf[pl.ds(h * 16, 16)] = exh
                rbuf[pl.ds(h * 16, 16)] = 1.0 / den
            degv = plsc.load_gather(
                facc, [lane, jnp.full((16,), 483, jnp.int32)]) + 1.0
            fdeg[...] = degv

            def frow(j, _):
                jf = jnp.full((16,), j, jnp.int32)
                for h in range(H):
                    hj = jnp.full((16,), h * 16 + j, jnp.int32)
                    exj = plsc.load_gather(ebuf, [hj])
                    rdj = plsc.load_gather(rbuf, [hj])
                    for k in range(C // 16):
                        col = h * C + k * 16
                        a = plsc.load_gather(facc, [jf, col + lane])
                        xlc = plsc.load_gather(xlb, [jf, col + lane])
                        val = (a + exj * xlc) * rdj + bgat_v[pl.ds(col, 16)]
                        plsc.store_scatter(fh1, [jf, col + lane],
                                           jnp.maximum(val, 0.0))
                return 0

            lax.fori_loop(0, 16, frow, 0)
            pltpu.sync_copy(fh1, h1_hbm.at[pl.ds(r0g, 16)])
            pltpu.sync_copy(fdeg, deg_hbm.at[pl.ds(r0g, 16)])
            return 0

        lax.fori_loop(0, RPS // 16, fblock, 0)
        plsc.subcore_barrier()
        return 0

    lax.fori_loop(0, NRANGE // 2, rpass, 0)


def _gcn_sc(src_hbm, dst_hbm, ew_hbm, hx_hbm, dinv_hbm, bg_hbm,
            h2_hbm,
            src_v, dst_v, ew_v, ssrc, sdst, sew, dinv_v, bg_v,
            idx_s, idx_w, cbuf, hxb, outb, facc, zrow, acc_sh, sem1):
    cid = lax.axis_index("c")
    sid = lax.axis_index("s")
    lane = _lane()
    pltpu.sync_copy(dinv_hbm, dinv_v)
    pltpu.sync_copy(bg_hbm, bg_v)
    _zero_block(zrow, HC)

    def rpass(rp, _):
        lo = (cid * (NRANGE // 2) + rp) * RANGE

        def clr(b, _):
            pltpu.sync_copy(zrow, acc_sh.at[pl.ds(sid * RPS + b * 16, 16)])
            return 0

        lax.fori_loop(0, RPS // 16, clr, 0)
        plsc.subcore_barrier()

        def eblock(blk, _):
            e0 = sid * EPS + blk * EBLK
            pltpu.sync_copy(src_hbm.at[pl.ds(e0, EBLK)], src_v)
            pltpu.sync_copy(dst_hbm.at[pl.ds(e0, EBLK)], dst_v)
            pltpu.sync_copy(ew_hbm.at[pl.ds(e0, EBLK)], ew_v)
            cnt = _compact_edges(src_v, dst_v, ew_v, ssrc, sdst, sew,
                                 lo, lo + RANGE)
            ngroups = (cnt + 15) >> 4

            def group(g, _):
                off = g * 16
                s16 = ssrc[pl.ds(off, 16)]
                d16 = sdst[pl.ds(off, 16)]
                w16 = sew[pl.ds(off, 16)]
                valid = (off + lane) < cnt
                idx_s[...] = s16
                cp1 = pltpu.async_copy(hx_hbm.at[idx_s], hxb, sem1)
                dsrc = plsc.load_gather(dinv_v, [s16])
                ddst = plsc.load_gather(dinv_v, [d16])
                coef = jnp.where(valid, dsrc * w16 * ddst, 0.0)
                cbuf[...] = coef
                cp1.wait()

                def edge(j, _):
                    jf = jnp.full((16,), j, jnp.int32)
                    cj = plsc.load_gather(cbuf,
                                          [jnp.full((16,), j, jnp.int32)])
                    for k in range(HC // 16):
                        col = k * 16
                        v = plsc.load_gather(hxb, [jf, col + lane])
                        plsc.store_scatter(outb, [jf, col + lane], v * cj)
                    return 0

                lax.fori_loop(0, 16, edge, 0)
                idx_w[...] = jnp.where(valid, d16 - lo, 0)
                pltpu.sync_copy(outb, acc_sh.at[idx_w], add=True)
                return 0

            lax.fori_loop(0, ngroups, group, 0)
            return 0

        lax.fori_loop(0, EPS // EBLK, eblock, 0)
        plsc.subcore_barrier()

        def fblock(b, _):
            r0l = sid * RPS + b * 16
            r0g = lo + r0l
            pltpu.sync_copy(hx_hbm.at[pl.ds(r0g, 16)], hxb)
            pltpu.sync_copy(acc_sh.at[pl.ds(r0l, 16)], facc)
            dself = plsc.load_gather(dinv_v, [r0g + lane])
            cbuf[...] = dself * dself

            def frow(j, _):
                jf = jnp.full((16,), j, jnp.int32)
                cj = plsc.load_gather(cbuf, [jnp.full((16,), j, jnp.int32)])
                for k in range(HC // 16):
                    col = k * 16
                    a = plsc.load_gather(facc, [jf, col + lane])
                    hv = plsc.load_gather(hxb, [jf, col + lane])
                    plsc.store_scatter(outb, [jf, col + lane],
                                       a + cj * hv + bg_v[pl.ds(col, 16)])
                return 0

            lax.fori_loop(0, 16, frow, 0)
            pltpu.sync_copy(outb, h2_hbm.at[pl.ds(r0g, 16)])
            return 0

        lax.fori_loop(0, RPS // 16, fblock, 0)
        plsc.subcore_barrier()
        return 0

    lax.fori_loop(0, NRANGE // 2, rpass, 0)


def _pool_sc(h2_hbm, batch_hbm, g_hbm,
             batch_v, rowb, sum_v, max_v, gbuf, sem1):
    cid = lax.axis_index("c")
    sid = lax.axis_index("s")
    lane = _lane()
    wid = cid * NSUB + sid
    pltpu.sync_copy(batch_hbm, batch_v)

    def count_less(gid):
        def body(i, acc):
            m = batch_v[pl.ds(i * 16, 16)] < gid
            return acc + plsc.all_reduce_population_count(m)
        acc = lax.fori_loop(0, N // 16, body, jnp.zeros((16,), jnp.int32))
        return jnp.max(acc)

    z = jnp.zeros((16,), jnp.float32)
    ninf = jnp.full((16,), -jnp.inf, jnp.float32)
    for gg in range(2):
        gid = wid * 2 + gg
        s0 = count_less(gid)
        s1 = count_less(gid + 1)
        for k in range(HC // 16):
            sum_v[pl.ds(k * 16, 16)] = z
            max_v[pl.ds(k * 16, 16)] = ninf
        nb = (s1 - s0 + 15) >> 4

        def block(ib, _):
            r0 = s0 + ib * 16
            pltpu.sync_copy(h2_hbm.at[pl.ds(r0, 16)], rowb)

            def row(j, _):
                jf = jnp.full((16,), j, jnp.int32)
                vr = jnp.full((16,), (r0 + j) < s1)
                for k in range(HC // 16):
                    ch = plsc.load_gather(rowb, [jf, k * 16 + lane])
                    sv = sum_v[pl.ds(k * 16, 16)]
                    sum_v[pl.ds(k * 16, 16)] = sv + jnp.where(vr, ch, 0.0)
                    mv = max_v[pl.ds(k * 16, 16)]
                    max_v[pl.ds(k * 16, 16)] = jnp.maximum(
                        mv, jnp.where(vr, ch, -jnp.inf))
                return 0

            lax.fori_loop(0, 16, row, 0)
            return 0

        lax.fori_loop(0, nb, block, 0)
        cntv = jnp.full((16,), s1 - s0, jnp.int32).astype(jnp.float32)
        rinv = 1.0 / jnp.maximum(cntv, 1.0)
        for k in range(HC // 16):
            gbuf[pl.ds(k * 16, 16)] = sum_v[pl.ds(k * 16, 16)] * rinv
            gbuf[pl.ds(HC + k * 16, 16)] = max_v[pl.ds(k * 16, 16)]
        pltpu.sync_copy(gbuf, g_hbm.at[gid])


_gat_call = pl.kernel(
    _gat_sc, mesh=_mesh,
    out_type=[jax.ShapeDtypeStruct((NPAD, HC), jnp.float32),
              jax.ShapeDtypeStruct((NPAD,), jnp.float32)],
    compiler_params=_sc_params,
    scratch_types=[
        pltpu.VMEM((EBLK,), jnp.int32), pltpu.VMEM((EBLK,), jnp.int32),
        pltpu.VMEM((EBLK,), jnp.float32),
        pltpu.VMEM((EBLK + 16,), jnp.int32),
        pltpu.VMEM((EBLK + 16,), jnp.int32),
        pltpu.VMEM((EBLK + 16,), jnp.float32),
        pltpu.VMEM((HC,), jnp.float32), pltpu.VMEM((HC,), jnp.float32),
        pltpu.VMEM((16,), jnp.int32), pltpu.VMEM((16,), jnp.int32),
        pltpu.VMEM((16,), jnp.int32),
        pltpu.VMEM((48,), jnp.float32), pltpu.VMEM((48,), jnp.float32),
        pltpu.VMEM((16, HC), jnp.float32), pltpu.VMEM((16, HC), jnp.float32),
        pltpu.VMEM((16, ACCW), jnp.float32),
        pltpu.VMEM((16, ACCW), jnp.float32),
        pltpu.VMEM((16, HC), jnp.float32),
        pltpu.VMEM((16,), jnp.float32),
        pltpu.VMEM((16, ACCW), jnp.float32),
        pltpu.VMEM_SHARED((RANGE, ACCW), jnp.float32),
        pltpu.SemaphoreType.DMA, pltpu.SemaphoreType.DMA,
    ])

_gcn_call = pl.kernel(
    _gcn_sc, mesh=_mesh,
    out_type=[jax.ShapeDtypeStruct((NPAD, HC), jnp.float32)],
    compiler_params=_sc_params,
    scratch_types=[
        pltpu.VMEM((EBLK,), jnp.int32), pltpu.VMEM((EBLK,), jnp.int32),
        pltpu.VMEM((EBLK,), jnp.float32),
        pltpu.VMEM((EBLK + 16,), jnp.int32),
        pltpu.VMEM((EBLK + 16,), jnp.int32),
        pltpu.VMEM((EBLK + 16,), jnp.float32),
        pltpu.VMEM((NPAD,), jnp.float32), pltpu.VMEM((HC,), jnp.float32),
        pltpu.VMEM((16,), jnp.int32), pltpu.VMEM((16,), jnp.int32),
        pltpu.VMEM((16,), jnp.float32),
        pltpu.VMEM((16, HC), jnp.float32), pltpu.VMEM((16, HC), jnp.float32),
        pltpu.VMEM((16, HC), jnp.float32), pltpu.VMEM((16, HC), jnp.float32),
        pltpu.VMEM_SHARED((RANGE, HC), jnp.float32),
        pltpu.SemaphoreType.DMA,
    ])

_pool_call = pl.kernel(
    _pool_sc, mesh=_mesh,
    out_type=[jax.ShapeDtypeStruct((B, 2 * HC), jnp.float32)],
    compiler_params=_sc_params,
    scratch_types=[
        pltpu.VMEM((N,), jnp.int32),
        pltpu.VMEM((16, HC), jnp.float32),
        pltpu.VMEM((HC,), jnp.float32), pltpu.VMEM((HC,), jnp.float32),
        pltpu.VMEM((2 * HC,), jnp.float32),
        pltpu.SemaphoreType.DMA,
    ])


def _lin2_tc(xin_ref, wl_ref, wr_ref, bl_ref, br_ref, xl_ref, xr_ref):
    xin = xin_ref[...]
    dn = (((1,), (1,)), ((), ()))
    xl_ref[...] = lax.dot_general(xin, wl_ref[...], dn,
                                  preferred_element_type=jnp.float32) + bl_ref[...]
    xr_ref[...] = lax.dot_general(xin, wr_ref[...], dn,
                                  preferred_element_type=jnp.float32) + br_ref[...]


def _gcnw_tc(h1_ref, wg_ref, deg_ref, hx_ref, dinv_ref):
    dn = (((1,), (1,)), ((), ()))
    hx_ref[...] = lax.dot_general(h1_ref[...], wg_ref[...], dn,
                                  preferred_element_type=jnp.float32)

    @pl.when(pl.program_id(0) == 0)
    def _():
        dinv_ref[...] = lax.rsqrt(deg_ref[...])


def _mlp_tc(g_ref, w1_ref, b1_ref, w2_ref, b2_ref, w3_ref, b3_ref, w4_ref,
            out_ref):
    dn = (((1,), (1,)), ((), ()))
    g = g_ref[...]
    g = jnp.maximum(lax.dot_general(g, w1_ref[...], dn,
                                    preferred_element_type=jnp.float32)
                    + b1_ref[...], 0.0)
    g = lax.dot_general(g, w2_ref[...], dn,
                        preferred_element_type=jnp.float32) + b2_ref[...]
    g = lax.dot_general(g, w3_ref[...], dn,
                        preferred_element_type=jnp.float32) + b3_ref[...]
    out_ref[...] = lax.dot_general(g, w4_ref[...], dn,
                                   preferred_element_type=jnp.float32)


_MB = 1024


def _lin2_call(xin, Wl, Wr, bl, br):
    grid = (NPAD // _MB,)
    return pl.pallas_call(
        _lin2_tc,
        grid=grid,
        in_specs=[
            pl.BlockSpec((_MB, 160), lambda i: (i, 0)),
            pl.BlockSpec((HC, 160), lambda i: (0, 0)),
            pl.BlockSpec((HC, 160), lambda i: (0, 0)),
            pl.BlockSpec((1, HC), lambda i: (0, 0)),
            pl.BlockSpec((1, HC), lambda i: (0, 0)),
        ],
        out_specs=[
            pl.BlockSpec((_MB, HC), lambda i: (i, 0)),
            pl.BlockSpec((_MB, HC), lambda i: (i, 0)),
        ],
        out_shape=[jax.ShapeDtypeStruct((NPAD, HC), jnp.float32),
                   jax.ShapeDtypeStruct((NPAD, HC), jnp.float32)],
    )(xin, Wl, Wr, bl, br)


def _gcnw_call(h1, Wg, deg2d):
    grid = (NPAD // _MB,)
    return pl.pallas_call(
        _gcnw_tc,
        grid=grid,
        in_specs=[
            pl.BlockSpec((_MB, HC), lambda i: (i, 0)),
            pl.BlockSpec((HC, HC), lambda i: (0, 0)),
            pl.BlockSpec((NPAD // 128, 128), lambda i: (0, 0)),
        ],
        out_specs=[
            pl.BlockSpec((_MB, HC), lambda i: (i, 0)),
            pl.BlockSpec((NPAD // 128, 128), lambda i: (0, 0)),
        ],
        out_shape=[jax.ShapeDtypeStruct((NPAD, HC), jnp.float32),
                   jax.ShapeDtypeStruct((NPAD // 128, 128), jnp.float32)],
    )(h1, Wg, deg2d)


def _mlp_call(g, W1, b1, W2, b2, W3, b3, W4):
    return pl.pallas_call(
        _mlp_tc,
        out_shape=jax.ShapeDtypeStruct((B, 1), jnp.float32),
    )(g, W1, b1.reshape(1, -1), W2, b2.reshape(1, -1), W3, b3.reshape(1, -1),
      W4)


def kernel(x, pe_enc, edge_index, edge_weight, batch, Wl, bl, Wr, br, att,
           b_gat, Wg, bg, W1, b1, W2, b2, W3, b3, W4):
    xin = jnp.concatenate([x, pe_enc], axis=1)
    xin = jnp.pad(xin, ((0, NPAD - N), (0, 0)))
    xl, xr = _lin2_call(xin, Wl, Wr, bl.reshape(1, -1), br.reshape(1, -1))
    src = edge_index[0]
    dst = edge_index[1]
    h1, deg = _gat_call(src, dst, edge_weight, xl, xr, att.reshape(-1), b_gat)
    hx, dinv2d = _gcnw_call(h1, Wg, deg.reshape(NPAD // 128, 128))
    h2 = _gcn_call(src, dst, edge_weight, hx, dinv2d.reshape(-1), bg)[0]
    g = _pool_call(h2, batch)[0]
    return _mlp_call(g, W1, b1, W2, b2, W3, b3, W4)


# speedup vs baseline: 1.2206x; 1.2206x over previous
import functools

import jax
import jax.numpy as jnp
from jax import lax
from jax.experimental import pallas as pl
from jax.experimental.pallas import tpu as pltpu
from jax.experimental.pallas import tpu_sc as plsc

N = 10000
E = 160000
H = 3
C = 160
HC = 480
B = 64
NPAD = 10240
NRANGE = 8
RANGE = NPAD // NRANGE
NSUB = 16
RPS = RANGE // NSUB
EPS = E // NSUB
EBLK = 2000
ACCW = 496

_mesh = plsc.VectorSubcoreMesh(core_axis_name="c", subcore_axis_name="s")
_sc_params = pltpu.CompilerParams(use_tc_tiling_on_sc=False,
                                  needs_layout_passes=False)


def _lane():
    return lax.iota(jnp.int32, 16)


def _att_evecs(xlb, xrb, att_v, lane):
    def row_body(j, es):
        jf = jnp.full((16,), j, jnp.int32)
        new = []
        for h in range(H):
            acc = jnp.zeros((16,), jnp.float32)
            for k in range(C // 16):
                col = h * C + k * 16
                xlc = plsc.load_gather(xlb, [jf, col + lane])
                xrc = plsc.load_gather(xrb, [jf, col + lane])
                s = xlc + xrc
                m = jnp.maximum(s, 0.2 * s)
                acc = acc + m * att_v[pl.ds(col, 16)]
            eh = jnp.sum(acc)
            new.append(jnp.where(lane == j, eh, es[h]))
        return tuple(new)

    z = jnp.zeros((16,), jnp.float32)
    return lax.fori_loop(0, 16, row_body, (z, z, z))


def _zero_block(ref, width):
    lane = _lane()
    z = jnp.zeros((16,), jnp.float32)

    def body(j, _):
        jf = jnp.full((16,), j, jnp.int32)
        for k in range(width // 16):
            plsc.store_scatter(ref, [jf, k * 16 + lane], z)
        return 0

    lax.fori_loop(0, 16, body, 0)


def _compact_edges(src_v, dst_v, ew_v, ssrc, sdst, sew, lo, hi):
    def body(i, cnt):
        off = i * 16
        s16 = src_v[pl.ds(off, 16)]
        d16 = dst_v[pl.ds(off, 16)]
        w16 = ew_v[pl.ds(off, 16)]
        m = (d16 >= lo) & (d16 < hi)
        plsc.store_compressed(ssrc.at[pl.ds(cnt, 16)], s16, mask=m)
        plsc.store_compressed(sdst.at[pl.ds(cnt, 16)], d16, mask=m)
        plsc.store_compressed(sew.at[pl.ds(cnt, 16)], w16, mask=m)
        pc = plsc.all_reduce_population_count(m)
        return cnt + jnp.max(pc)

    cnt = lax.fori_loop(0, EBLK // 16, body, jnp.int32(0))
    ssrc[pl.ds(cnt, 16)] = jnp.zeros((16,), jnp.int32)
    sdst[pl.ds(cnt, 16)] = jnp.full((16,), lo, jnp.int32)
    sew[pl.ds(cnt, 16)] = jnp.zeros((16,), jnp.float32)
    return cnt


def _gat_sc(src_hbm, dst_hbm, ew_hbm, xl_hbm, xr_hbm, att_hbm, bgat_hbm,
            h1_hbm, deg_hbm,
            src_v, dst_v, ew_v, ssrc, sdst, sew, att_v, bgat_v,
            idx_s0, idx_s1, idx_d0, idx_d1, idx_w, ebuf, rbuf,
            xlb0, xlb1, xrb0, xrb1, outb, facc, fh1,
            fdeg, zrow, acc_sh, semA0, semA1, semB0, semB1):
    xlbs = (xlb0, xlb1)
    xrbs = (xrb0, xrb1)
    idss = (idx_s0, idx_s1)
    idds = (idx_d0, idx_d1)
    semas = (semA0, semA1)
    sembs = (semB0, semB1)
    xlb = xlb0
    xrb = xrb0
    cid = lax.axis_index("c")
    sid = lax.axis_index("s")
    lane = _lane()
    pltpu.sync_copy(att_hbm, att_v)
    pltpu.sync_copy(bgat_hbm, bgat_v)
    _zero_block(zrow, ACCW)
    z = jnp.zeros((16,), jnp.float32)
    for c in range(484, ACCW):
        plsc.store_scatter(outb, [lane, jnp.full((16,), c, jnp.int32)], z)

    def rpass(rp, _):
        lo = (cid * (NRANGE // 2) + rp) * RANGE

        def clr(b, _):
            pltpu.sync_copy(zrow, acc_sh.at[pl.ds(sid * RPS + b * 16, 16)])
            return 0

        lax.fori_loop(0, RPS // 16, clr, 0)
        plsc.subcore_barrier()

        def eblock(blk, _):
            e0 = sid * EPS + blk * EBLK
            pltpu.sync_copy(src_hbm.at[pl.ds(e0, EBLK)], src_v)
            pltpu.sync_copy(dst_hbm.at[pl.ds(e0, EBLK)], dst_v)
            pltpu.sync_copy(ew_hbm.at[pl.ds(e0, EBLK)], ew_v)
            cnt = _compact_edges(src_v, dst_v, ew_v, ssrc, sdst, sew,
                                 lo, lo + RANGE)
            ngroups = (cnt + 15) >> 4

            def issue(g, b):
                @pl.when(g < ngroups)
                def _():
                    off = g * 16
                    idss[b][...] = ssrc[pl.ds(off, 16)]
                    idds[b][...] = sdst[pl.ds(off, 16)]
                    pltpu.async_copy(xl_hbm.at[idss[b]], xlbs[b], semas[b])
                    pltpu.async_copy(xr_hbm.at[idds[b]], xrbs[b], sembs[b])

            issue(jnp.int32(0), 0)
            issue(jnp.int32(1), 1)

            def gpair(p, _):
                for b in range(2):
                    g = p * 2 + b

                    @pl.when(g < ngroups)
                    def _():
                        pltpu.make_async_copy(xl_hbm.at[idss[b]], xlbs[b],
                                              semas[b]).wait()
                        pltpu.make_async_copy(xr_hbm.at[idds[b]], xrbs[b],
                                              sembs[b]).wait()
                        off = g * 16
                        d16 = sdst[pl.ds(off, 16)]
                        w16 = sew[pl.ds(off, 16)]
                        valid = (off + lane) < cnt
                        ev = _att_evecs(xlbs[b], xrbs[b], att_v, lane)
                        vf = jnp.where(valid, 1.0, 0.0).astype(jnp.float32)
                        exs = [jnp.exp(ev[h]) * vf for h in range(H)]
                        for h in range(H):
                            ebuf[pl.ds(h * 16, 16)] = exs[h]
                            plsc.store_scatter(
                                outb,
                                [lane, jnp.full((16,), 480 + h, jnp.int32)],
                                exs[h])
                        plsc.store_scatter(
                            outb, [lane, jnp.full((16,), 483, jnp.int32)],
                            w16 * vf)

                        def edge(j, _):
                            jf = jnp.full((16,), j, jnp.int32)
                            for h in range(H):
                                exj = plsc.load_gather(
                                    ebuf,
                                    [jnp.full((16,), h * 16 + j, jnp.int32)])
                                for k in range(C // 16):
                                    col = h * C + k * 16
                                    xlc = plsc.load_gather(
                                        xlbs[b], [jf, col + lane])
                                    plsc.store_scatter(outb, [jf, col + lane],
                                                       xlc * exj)
                            return 0

                        lax.fori_loop(0, 16, edge, 0)
                        idx_w[...] = jnp.where(valid, d16 - lo, 0)
                        pltpu.sync_copy(outb, acc_sh.at[idx_w], add=True)
                        issue(g + 2, b)
                return 0

            lax.fori_loop(0, (ngroups + 1) >> 1, gpair, 0)
            return 0

        lax.fori_loop(0, EPS // EBLK, eblock, 0)
        plsc.subcore_barrier()

        def fblock(b, _):
            r0l = sid * RPS + b * 16
            r0g = lo + r0l
            pltpu.sync_copy(xl_hbm.at[pl.ds(r0g, 16)], xlb)
            pltpu.sync_copy(xr_hbm.at[pl.ds(r0g, 16)], xrb)
            pltpu.sync_copy(acc_sh.at[pl.ds(r0l, 16)], facc)
            ev = _att_evecs(xlb, xrb, att_v, lane)
            for h in range(H):
                exh = jnp.exp(ev[h])
                den = plsc.load_gather(
                    facc, [lane, jnp.full((16,), 480 + h, jnp.int32)]) + exh
                ebuf[pl.ds(h * 16, 16)] = exh
                rbuf[pl.ds(h * 16, 16)] = 1.0 / den
            degv = plsc.load_gather(
                facc, [lane, jnp.full((16,), 483, jnp.int32)]) + 1.0
            fdeg[...] = degv

            def frow(j, _):
                jf = jnp.full((16,), j, jnp.int32)
                for h in range(H):
                    hj = jnp.full((16,), h * 16 + j, jnp.int32)
                    exj = plsc.load_gather(ebuf, [hj])
                    rdj = plsc.load_gather(rbuf, [hj])
                    for k in range(C // 16):
                        col = h * C + k * 16
                        a = plsc.load_gather(facc, [jf, col + lane])
                        xlc = plsc.load_gather(xlb, [jf, col + lane])
                        val = (a + exj * xlc) * rdj + bgat_v[pl.ds(col, 16)]
                        plsc.store_scatter(fh1, [jf, col + lane],
                                           jnp.maximum(val, 0.0))
                return 0

            lax.fori_loop(0, 16, frow, 0)
            pltpu.sync_copy(fh1, h1_hbm.at[pl.ds(r0g, 16)])
            pltpu.sync_copy(fdeg, deg_hbm.at[pl.ds(r0g, 16)])
            return 0

        lax.fori_loop(0, RPS // 16, fblock, 0)
        plsc.subcore_barrier()
        return 0

    lax.fori_loop(0, NRANGE // 2, rpass, 0)


def _gcn_sc(src_hbm, dst_hbm, ew_hbm, hx_hbm, dinv_hbm, bg_hbm,
            h2_hbm,
            src_v, dst_v, ew_v, ssrc, sdst, sew, dinv_v, bg_v,
            idx_s0, idx_s1, idx_w, cbuf, hxb0, hxb1, outb, facc, zrow,
            acc_sh, semA0, semA1):
    hxbs = (hxb0, hxb1)
    idss = (idx_s0, idx_s1)
    semas = (semA0, semA1)
    hxb = hxb0
    cid = lax.axis_index("c")
    sid = lax.axis_index("s")
    lane = _lane()
    pltpu.sync_copy(dinv_hbm, dinv_v)
    pltpu.sync_copy(bg_hbm, bg_v)
    _zero_block(zrow, HC)

    def rpass(rp, _):
        lo = (cid * (NRANGE // 2) + rp) * RANGE

        def clr(b, _):
            pltpu.sync_copy(zrow, acc_sh.at[pl.ds(sid * RPS + b * 16, 16)])
            return 0

        lax.fori_loop(0, RPS // 16, clr, 0)
        plsc.subcore_barrier()

        def eblock(blk, _):
            e0 = sid * EPS + blk * EBLK
            pltpu.sync_copy(src_hbm.at[pl.ds(e0, EBLK)], src_v)
            pltpu.sync_copy(dst_hbm.at[pl.ds(e0, EBLK)], dst_v)
            pltpu.sync_copy(ew_hbm.at[pl.ds(e0, EBLK)], ew_v)
            cnt = _compact_edges(src_v, dst_v, ew_v, ssrc, sdst, sew,
                                 lo, lo + RANGE)
            ngroups = (cnt + 15) >> 4

            def issue(g, b):
                @pl.when(g < ngroups)
                def _():
                    off = g * 16
                    idss[b][...] = ssrc[pl.ds(off, 16)]
                    pltpu.async_copy(hx_hbm.at[idss[b]], hxbs[b], semas[b])

            issue(jnp.int32(0), 0)
            issue(jnp.int32(1), 1)

            def gpair(p, _):
                for b in range(2):
                    g = p * 2 + b

                    @pl.when(g < ngroups)
                    def _():
                        pltpu.make_async_copy(hx_hbm.at[idss[b]], hxbs[b],
                                              semas[b]).wait()
                        off = g * 16
                        s16 = ssrc[pl.ds(off, 16)]
                        d16 = sdst[pl.ds(off, 16)]
                        w16 = sew[pl.ds(off, 16)]
                        valid = (off + lane) < cnt
                        dsrc = plsc.load_gather(dinv_v, [s16])
                        ddst = plsc.load_gather(dinv_v, [d16])
                        coef = jnp.where(valid, dsrc * w16 * ddst, 0.0)
                        cbuf[...] = coef

                        def edge(j, _):
                            jf = jnp.full((16,), j, jnp.int32)
                            cj = plsc.load_gather(
                                cbuf, [jnp.full((16,), j, jnp.int32)])
                            for k in range(HC // 16):
                                col = k * 16
                                v = plsc.load_gather(hxbs[b],
                                                     [jf, col + lane])
                                plsc.store_scatter(outb, [jf, col + lane],
                                                   v * cj)
                            return 0

                        lax.fori_loop(0, 16, edge, 0)
                        idx_w[...] = jnp.where(valid, d16 - lo, 0)
                        pltpu.sync_copy(outb, acc_sh.at[idx_w], add=True)
                        issue(g + 2, b)
                return 0

            lax.fori_loop(0, (ngroups + 1) >> 1, gpair, 0)
            return 0

        lax.fori_loop(0, EPS // EBLK, eblock, 0)
        plsc.subcore_barrier()

        def fblock(b, _):
            r0l = sid * RPS + b * 16
            r0g = lo + r0l
            pltpu.sync_copy(hx_hbm.at[pl.ds(r0g, 16)], hxb)
            pltpu.sync_copy(acc_sh.at[pl.ds(r0l, 16)], facc)
            dself = plsc.load_gather(dinv_v, [r0g + lane])
            cbuf[...] = dself * dself

            def frow(j, _):
                jf = jnp.full((16,), j, jnp.int32)
                cj = plsc.load_gather(cbuf, [jnp.full((16,), j, jnp.int32)])
                for k in range(HC // 16):
                    col = k * 16
                    a = plsc.load_gather(facc, [jf, col + lane])
                    hv = plsc.load_gather(hxb, [jf, col + lane])
                    plsc.store_scatter(outb, [jf, col + lane],
                                       a + cj * hv + bg_v[pl.ds(col, 16)])
                return 0

            lax.fori_loop(0, 16, frow, 0)
            pltpu.sync_copy(outb, h2_hbm.at[pl.ds(r0g, 16)])
            return 0

        lax.fori_loop(0, RPS // 16, fblock, 0)
        plsc.subcore_barrier()
        return 0

    lax.fori_loop(0, NRANGE // 2, rpass, 0)


def _pool_sc(h2_hbm, batch_hbm, g_hbm,
             batch_v, rowb, sum_v, max_v, gbuf, sem1):
    cid = lax.axis_index("c")
    sid = lax.axis_index("s")
    lane = _lane()
    wid = cid * NSUB + sid
    pltpu.sync_copy(batch_hbm, batch_v)

    def count_less(gid):
        def body(i, acc):
            m = batch_v[pl.ds(i * 16, 16)] < gid
            return acc + plsc.all_reduce_population_count(m)
        acc = lax.fori_loop(0, N // 16, body, jnp.zeros((16,), jnp.int32))
        return jnp.max(acc)

    z = jnp.zeros((16,), jnp.float32)
    ninf = jnp.full((16,), -jnp.inf, jnp.float32)
    for gg in range(2):
        gid = wid * 2 + gg
        s0 = count_less(gid)
        s1 = count_less(gid + 1)
        for k in range(HC // 16):
            sum_v[pl.ds(k * 16, 16)] = z
            max_v[pl.ds(k * 16, 16)] = ninf
        nb = (s1 - s0 + 15) >> 4

        def block(ib, _):
            r0 = s0 + ib * 16
            pltpu.sync_copy(h2_hbm.at[pl.ds(r0, 16)], rowb)

            def row(j, _):
                jf = jnp.full((16,), j, jnp.int32)
                vr = jnp.full((16,), (r0 + j) < s1)
                for k in range(HC // 16):
                    ch = plsc.load_gather(rowb, [jf, k * 16 + lane])
                    sv = sum_v[pl.ds(k * 16, 16)]
                    sum_v[pl.ds(k * 16, 16)] = sv + jnp.where(vr, ch, 0.0)
                    mv = max_v[pl.ds(k * 16, 16)]
                    max_v[pl.ds(k * 16, 16)] = jnp.maximum(
                        mv, jnp.where(vr, ch, -jnp.inf))
                return 0

            lax.fori_loop(0, 16, row, 0)
            return 0

        lax.fori_loop(0, nb, block, 0)
        cntv = jnp.full((16,), s1 - s0, jnp.int32).astype(jnp.float32)
        rinv = 1.0 / jnp.maximum(cntv, 1.0)
        for k in range(HC // 16):
            gbuf[pl.ds(k * 16, 16)] = sum_v[pl.ds(k * 16, 16)] * rinv
            gbuf[pl.ds(HC + k * 16, 16)] = max_v[pl.ds(k * 16, 16)]
        pltpu.sync_copy(gbuf, g_hbm.at[gid])


_gat_call = pl.kernel(
    _gat_sc, mesh=_mesh,
    out_type=[jax.ShapeDtypeStruct((NPAD, HC), jnp.float32),
              jax.ShapeDtypeStruct((NPAD,), jnp.float32)],
    compiler_params=_sc_params,
    scratch_types=[
        pltpu.VMEM((EBLK,), jnp.int32), pltpu.VMEM((EBLK,), jnp.int32),
        pltpu.VMEM((EBLK,), jnp.float32),
        pltpu.VMEM((EBLK + 16,), jnp.int32),
        pltpu.VMEM((EBLK + 16,), jnp.int32),
        pltpu.VMEM((EBLK + 16,), jnp.float32),
        pltpu.VMEM((HC,), jnp.float32), pltpu.VMEM((HC,), jnp.float32),
        pltpu.VMEM((16,), jnp.int32), pltpu.VMEM((16,), jnp.int32),
        pltpu.VMEM((16,), jnp.int32), pltpu.VMEM((16,), jnp.int32),
        pltpu.VMEM((16,), jnp.int32),
        pltpu.VMEM((48,), jnp.float32), pltpu.VMEM((48,), jnp.float32),
        pltpu.VMEM((16, HC), jnp.float32), pltpu.VMEM((16, HC), jnp.float32),
        pltpu.VMEM((16, HC), jnp.float32), pltpu.VMEM((16, HC), jnp.float32),
        pltpu.VMEM((16, ACCW), jnp.float32),
        pltpu.VMEM((16, ACCW), jnp.float32),
        pltpu.VMEM((16, HC), jnp.float32),
        pltpu.VMEM((16,), jnp.float32),
        pltpu.VMEM((16, ACCW), jnp.float32),
        pltpu.VMEM_SHARED((RANGE, ACCW), jnp.float32),
        pltpu.SemaphoreType.DMA, pltpu.SemaphoreType.DMA,
        pltpu.SemaphoreType.DMA, pltpu.SemaphoreType.DMA,
    ])

_gcn_call = pl.kernel(
    _gcn_sc, mesh=_mesh,
    out_type=[jax.ShapeDtypeStruct((NPAD, HC), jnp.float32)],
    compiler_params=_sc_params,
    scratch_types=[
        pltpu.VMEM((EBLK,), jnp.int32), pltpu.VMEM((EBLK,), jnp.int32),
        pltpu.VMEM((EBLK,), jnp.float32),
        pltpu.VMEM((EBLK + 16,), jnp.int32),
        pltpu.VMEM((EBLK + 16,), jnp.int32),
        pltpu.VMEM((EBLK + 16,), jnp.float32),
        pltpu.VMEM((NPAD,), jnp.float32), pltpu.VMEM((HC,), jnp.float32),
        pltpu.VMEM((16,), jnp.int32), pltpu.VMEM((16,), jnp.int32),
        pltpu.VMEM((16,), jnp.int32),
        pltpu.VMEM((16,), jnp.float32),
        pltpu.VMEM((16, HC), jnp.float32), pltpu.VMEM((16, HC), jnp.float32),
        pltpu.VMEM((16, HC), jnp.float32), pltpu.VMEM((16, HC), jnp.float32),
        pltpu.VMEM((16, HC), jnp.float32),
        pltpu.VMEM_SHARED((RANGE, HC), jnp.float32),
        pltpu.SemaphoreType.DMA, pltpu.SemaphoreType.DMA,
    ])

_pool_call = pl.kernel(
    _pool_sc, mesh=_mesh,
    out_type=[jax.ShapeDtypeStruct((B, 2 * HC), jnp.float32)],
    compiler_params=_sc_params,
    scratch_types=[
        pltpu.VMEM((N,), jnp.int32),
        pltpu.VMEM((16, HC), jnp.float32),
        pltpu.VMEM((HC,), jnp.float32), pltpu.VMEM((HC,), jnp.float32),
        pltpu.VMEM((2 * HC,), jnp.float32),
        pltpu.SemaphoreType.DMA,
    ])


def _lin2_tc(xin_ref, wl_ref, wr_ref, bl_ref, br_ref, xl_ref, xr_ref):
    xin = xin_ref[...]
    dn = (((1,), (1,)), ((), ()))
    xl_ref[...] = lax.dot_general(xin, wl_ref[...], dn,
                                  preferred_element_type=jnp.float32) + bl_ref[...]
    xr_ref[...] = lax.dot_general(xin, wr_ref[...], dn,
                                  preferred_element_type=jnp.float32) + br_ref[...]


def _gcnw_tc(h1_ref, wg_ref, deg_ref, hx_ref, dinv_ref):
    dn = (((1,), (1,)), ((), ()))
    hx_ref[...] = lax.dot_general(h1_ref[...], wg_ref[...], dn,
                                  preferred_element_type=jnp.float32)

    @pl.when(pl.program_id(0) == 0)
    def _():
        dinv_ref[...] = lax.rsqrt(deg_ref[...])


def _mlp_tc(g_ref, w1_ref, b1_ref, w2_ref, b2_ref, w3_ref, b3_ref, w4_ref,
            out_ref):
    dn = (((1,), (1,)), ((), ()))
    g = g_ref[...]
    g = jnp.maximum(lax.dot_general(g, w1_ref[...], dn,
                                    preferred_element_type=jnp.float32)
                    + b1_ref[...], 0.0)
    g = lax.dot_general(g, w2_ref[...], dn,
                        preferred_element_type=jnp.float32) + b2_ref[...]
    g = lax.dot_general(g, w3_ref[...], dn,
                        preferred_element_type=jnp.float32) + b3_ref[...]
    out_ref[...] = lax.dot_general(g, w4_ref[...], dn,
                                   preferred_element_type=jnp.float32)


_MB = 1024


def _lin2_call(xin, Wl, Wr, bl, br):
    grid = (NPAD // _MB,)
    return pl.pallas_call(
        _lin2_tc,
        grid=grid,
        in_specs=[
            pl.BlockSpec((_MB, 160), lambda i: (i, 0)),
            pl.BlockSpec((HC, 160), lambda i: (0, 0)),
            pl.BlockSpec((HC, 160), lambda i: (0, 0)),
            pl.BlockSpec((1, HC), lambda i: (0, 0)),
            pl.BlockSpec((1, HC), lambda i: (0, 0)),
        ],
        out_specs=[
            pl.BlockSpec((_MB, HC), lambda i: (i, 0)),
            pl.BlockSpec((_MB, HC), lambda i: (i, 0)),
        ],
        out_shape=[jax.ShapeDtypeStruct((NPAD, HC), jnp.float32),
                   jax.ShapeDtypeStruct((NPAD, HC), jnp.float32)],
    )(xin, Wl, Wr, bl, br)


def _gcnw_call(h1, Wg, deg2d):
    grid = (NPAD // _MB,)
    return pl.pallas_call(
        _gcnw_tc,
        grid=grid,
        in_specs=[
            pl.BlockSpec((_MB, HC), lambda i: (i, 0)),
            pl.BlockSpec((HC, HC), lambda i: (0, 0)),
            pl.BlockSpec((NPAD // 128, 128), lambda i: (0, 0)),
        ],
        out_specs=[
            pl.BlockSpec((_MB, HC), lambda i: (i, 0)),
            pl.BlockSpec((NPAD // 128, 128), lambda i: (0, 0)),
        ],
        out_shape=[jax.ShapeDtypeStruct((NPAD, HC), jnp.float32),
                   jax.ShapeDtypeStruct((NPAD // 128, 128), jnp.float32)],
    )(h1, Wg, deg2d)


def _mlp_call(g, W1, b1, W2, b2, W3, b3, W4):
    return pl.pallas_call(
        _mlp_tc,
        out_shape=jax.ShapeDtypeStruct((B, 1), jnp.float32),
    )(g, W1, b1.reshape(1, -1), W2, b2.reshape(1, -1), W3, b3.reshape(1, -1),
      W4)


def kernel(x, pe_enc, edge_index, edge_weight, batch, Wl, bl, Wr, br, att,
           b_gat, Wg, bg, W1, b1, W2, b2, W3, b3, W4):
    xin = jnp.concatenate([x, pe_enc], axis=1)
    xin = jnp.pad(xin, ((0, NPAD - N), (0, 0)))
    xl, xr = _lin2_call(xin, Wl, Wr, bl.reshape(1, -1), br.reshape(1, -1))
    src = edge_index[0]
    dst = edge_index[1]
    h1, deg = _gat_call(src, dst, edge_weight, xl, xr, att.reshape(-1), b_gat)
    hx, dinv2d = _gcnw_call(h1, Wg, deg.reshape(NPAD // 128, 128))
    h2 = _gcn_call(src, dst, edge_weight, hx, dinv2d.reshape(-1), bg)[0]
    g = _pool_call(h2, batch)[0]
    return _mlp_call(g, W1, b1, W2, b2, W3, b3, W4)
